# baseline (speedup 1.0000x reference)
import functools

import jax
import jax.numpy as jnp
from jax import lax
from jax.experimental import pallas as pl
from jax.experimental.pallas import tpu as pltpu
from jax.experimental.pallas import tpu_sc as plsc

_EMB = 64
_B = 16384
_NC = 2
_NS = 16
_NW = _NC * _NS
_BPW = _B // _NW
_BLKR = 32
_NBLK = _BPW // _BLKR
_DEPTH = 16


def _sc_gather(uid_ref, iid_ref, tabu_ref, tabi_ref, ub_ref, ib_ref,
               x_out, bs_out,
               uix_v, iix_v, stu_v, sti_v, sub_v, sib_v, x_w, bs_w,
               sem_u, sem_i, sem_ub, sem_ib):
    bpw = uix_v.shape[0]
    nblk = bpw // _BLKR
    wid = lax.axis_index("s") * _NC + lax.axis_index("c")
    base = wid * bpw

    for j in range(bpw // 128):
        sl = pl.ds(base + j * 128, 128)
        pltpu.sync_copy(uid_ref.at[sl], uix_v.at[pl.ds(j * 128, 128)])
        pltpu.sync_copy(iid_ref.at[sl], iix_v.at[pl.ds(j * 128, 128)])

    lanes = lax.iota(jnp.int32, 16)

    def drain_row(r):
        d16 = pl.ds(r * 16, 16)
        pltpu.make_async_copy(tabu_ref.at[:, :, pl.ds(0, 16)],
                              stu_v.at[:, :, d16], sem_u).wait()
        pltpu.make_async_copy(tabi_ref.at[:, :, pl.ds(0, 16)],
                              sti_v.at[:, :, d16], sem_i).wait()
        pltpu.make_async_copy(ub_ref.at[pl.ds(0, 16)],
                              sub_v.at[d16], sem_ub).wait()
        pltpu.make_async_copy(ib_ref.at[pl.ds(0, 16)],
                              sib_v.at[d16], sem_ib).wait()

    def block_body(b, carry):
        bs32 = pl.ds(base + b * _BLKR, _BLKR)

        def issue(r, carry2):
            g = (r // 16) * 16
            lane = r % 16
            uvec = uix_v[pl.ds(b * _BLKR + g, 16)]
            ivec = iix_v[pl.ds(b * _BLKR + g, 16)]
            urid = jnp.sum(jnp.where(lanes == lane, uvec, 0))
            irid = jnp.sum(jnp.where(lanes == lane, ivec, 0))
            ua = pl.multiple_of((urid >> 4) << 4, 16)
            ia = pl.multiple_of((irid >> 4) << 4, 16)
            r16 = pl.multiple_of(r * 16, 16)
            d16 = pl.ds(r16, 16)
            pltpu.async_copy(tabu_ref.at[:, :, pl.ds(ua, 16)],
                             stu_v.at[:, :, d16], sem_u)
            pltpu.async_copy(tabi_ref.at[:, :, pl.ds(ia, 16)],
                             sti_v.at[:, :, d16], sem_i)
            pltpu.async_copy(ub_ref.at[pl.ds(ua, 16)],
                             sub_v.at[d16], sem_ub)
            pltpu.async_copy(ib_ref.at[pl.ds(ia, 16)],
                             sib_v.at[d16], sem_ib)

            @pl.when(r >= _DEPTH)
            def _():
                drain_row(r - _DEPTH)

            return carry2

        lax.fori_loop(0, _BLKR, issue, 0)

        def tail(tr, carry2):
            drain_row(_BLKR - _DEPTH + tr)
            return carry2

        lax.fori_loop(0, _DEPTH, tail, 0)

        for g in range(_BLKR // 16):
            rvec = lanes + g * 16
            uvec = uix_v[pl.ds(b * _BLKR + g * 16, 16)]
            ivec = iix_v[pl.ds(b * _BLKR + g * 16, 16)]
            upos = rvec * 16 + (uvec & 15)
            ipos = rvec * 16 + (ivec & 15)
            for c in range(_EMB):
                cv = jnp.full((16,), c, jnp.int32)
                vu = plsc.load_gather(
                    stu_v, [cv >> 3, cv & 7, upos])
                plsc.store_scatter(x_w, [rvec, cv], vu)
                vi = plsc.load_gather(
                    sti_v, [cv >> 3, cv & 7, ipos])
                plsc.store_scatter(x_w, [rvec, cv + _EMB], vi)
            bs_w[pl.ds(g * 16, 16)] = (plsc.load_gather(sub_v, [upos])
                                       + plsc.load_gather(sib_v, [ipos]))

        pltpu.sync_copy(x_w, x_out.at[bs32])
        pltpu.sync_copy(bs_w, bs_out.at[bs32])
        return carry

    lax.fori_loop(0, nblk, block_body, 0)


def _make_sc_call(n):
    return functools.partial(
        pl.kernel,
        mesh=plsc.VectorSubcoreMesh(core_axis_name="c", subcore_axis_name="s"),
        compiler_params=pltpu.CompilerParams(needs_layout_passes=False),
        out_type=[
            jax.ShapeDtypeStruct((n, 2 * _EMB), jnp.float32),
            jax.ShapeDtypeStruct((n,), jnp.float32),
        ],
        scratch_types=[
            pltpu.VMEM((n // _NW,), jnp.int32),
            pltpu.VMEM((n // _NW,), jnp.int32),
            pltpu.VMEM((8, 8, 16 * _BLKR), jnp.float32),
            pltpu.VMEM((8, 8, 16 * _BLKR), jnp.float32),
            pltpu.VMEM((16 * _BLKR,), jnp.float32),
            pltpu.VMEM((16 * _BLKR,), jnp.float32),
            pltpu.VMEM((_BLKR, 2 * _EMB), jnp.float32),
            pltpu.VMEM((_BLKR,), jnp.float32),
            pltpu.SemaphoreType.DMA,
            pltpu.SemaphoreType.DMA,
            pltpu.SemaphoreType.DMA,
            pltpu.SemaphoreType.DMA,
        ],
    )(_sc_gather)


_BLK = 2048


def _mlp_body(x_ref, bs_ref, w1_ref, b1_ref, w2t_ref, b2_ref, w3r_ref,
              gb3_ref, out_ref):
    h1 = jnp.maximum(
        jnp.dot(x_ref[...], w1_ref[...], preferred_element_type=jnp.float32)
        + b1_ref[...], 0.0)
    h2 = jnp.maximum(
        lax.dot_general(h1, w2t_ref[...], (((1,), (1,)), ((), ())),
                        preferred_element_type=jnp.float32)
        + b2_ref[...], 0.0)
    m = jnp.sum(h2 * w3r_ref[...], axis=1)
    out_ref[...] = m + bs_ref[...] + gb3_ref[...]


def _mlp(x, bs, w1, b1, w2t, b2, w3r, gb3):
    grid = (x.shape[0] // _BLK,)
    return pl.pallas_call(
        _mlp_body,
        grid=grid,
        in_specs=[
            pl.BlockSpec((_BLK, 2 * _EMB), lambda i: (i, 0)),
            pl.BlockSpec((_BLK,), lambda i: (i,)),
            pl.BlockSpec((2 * _EMB, 128), lambda i: (0, 0)),
            pl.BlockSpec((128,), lambda i: (0,)),
            pl.BlockSpec((_EMB, 128), lambda i: (0, 0)),
            pl.BlockSpec((_EMB,), lambda i: (0,)),
            pl.BlockSpec((1, _EMB), lambda i: (0, 0)),
            pl.BlockSpec((1,), lambda i: (0,)),
        ],
        out_specs=pl.BlockSpec((_BLK,), lambda i: (i,)),
        out_shape=jax.ShapeDtypeStruct((x.shape[0],), jnp.float32),
    )(x, bs, w1, b1, w2t, b2, w3r, gb3)


def kernel(user_ids, item_ids, user_emb, item_emb, user_bias_t, item_bias_t,
           global_bias, W1, b1, W2, b2, W3, b3):
    uid = user_ids.astype(jnp.int32)
    iid = item_ids.astype(jnp.int32)
    tabu = user_emb.T.reshape(8, 8, 1000000)
    tabi = item_emb.T.reshape(8, 8, 1000000)

    ub = user_bias_t.reshape(-1)
    ib = item_bias_t.reshape(-1)
    w2t = W2.T
    w3r = W3.reshape(1, -1)
    gb3 = global_bias + b3

    half = _B // 2
    call = _make_sc_call(half)
    outs = []
    for h in range(2):
        hs = slice(h * half, (h + 1) * half)
        x, bs = call(uid[hs], iid[hs], tabu, tabi, ub, ib)
        outs.append(_mlp(x, bs, W1, b1, w2t, b2, w3r, gb3))
    return jnp.concatenate(outs)


# speedup vs baseline: 1.0505x; 1.0505x over previous
import functools

import jax
import jax.numpy as jnp
from jax import lax
from jax.experimental import pallas as pl
from jax.experimental.pallas import tpu as pltpu
from jax.experimental.pallas import tpu_sc as plsc

_EMB = 64
_B = 16384
_NC = 2
_NS = 16
_NW = _NC * _NS
_BPW = _B // _NW
_BLKR = 32
_NBLK = _BPW // _BLKR
_DEPTH = 16


def _sc_gather(uid_ref, iid_ref, tabu_ref, tabi_ref, ub_ref, ib_ref,
               x_out, bs_out,
               uix_v, iix_v, stu_v, sti_v, sub_v, sib_v, x_w, bs_w,
               sem_u, sem_i, sem_ub, sem_ib, sem_ox, sem_ob):
    bpw = uix_v.shape[0]
    nblk = bpw // _BLKR
    wid = lax.axis_index("s") * _NC + lax.axis_index("c")
    base = wid * bpw

    for j in range(bpw // 128):
        sl = pl.ds(base + j * 128, 128)
        pltpu.sync_copy(uid_ref.at[sl], uix_v.at[pl.ds(j * 128, 128)])
        pltpu.sync_copy(iid_ref.at[sl], iix_v.at[pl.ds(j * 128, 128)])

    lanes = lax.iota(jnp.int32, 16)

    def drain_row(r):
        d16 = pl.ds(r * 16, 16)
        pltpu.make_async_copy(tabu_ref.at[:, :, pl.ds(0, 16)],
                              stu_v.at[:, :, d16], sem_u).wait()
        pltpu.make_async_copy(tabi_ref.at[:, :, pl.ds(0, 16)],
                              sti_v.at[:, :, d16], sem_i).wait()
        pltpu.make_async_copy(ub_ref.at[pl.ds(0, 16)],
                              sub_v.at[d16], sem_ub).wait()
        pltpu.make_async_copy(ib_ref.at[pl.ds(0, 16)],
                              sib_v.at[d16], sem_ib).wait()

    def block_body(b, carry):
        bs32 = pl.ds(base + b * _BLKR, _BLKR)
        off = (b % 2) * _BLKR

        @pl.when(b >= 2)
        def _():
            pltpu.make_async_copy(x_w.at[pl.ds(off, _BLKR)],
                                  x_out.at[bs32], sem_ox).wait()
            pltpu.make_async_copy(bs_w.at[pl.ds(off, _BLKR)],
                                  bs_out.at[bs32], sem_ob).wait()

        def issue(r, carry2):
            g = (r // 16) * 16
            lane = r % 16
            uvec = uix_v[pl.ds(b * _BLKR + g, 16)]
            ivec = iix_v[pl.ds(b * _BLKR + g, 16)]
            urid = jnp.sum(jnp.where(lanes == lane, uvec, 0))
            irid = jnp.sum(jnp.where(lanes == lane, ivec, 0))
            ua = pl.multiple_of((urid >> 4) << 4, 16)
            ia = pl.multiple_of((irid >> 4) << 4, 16)
            r16 = pl.multiple_of(r * 16, 16)
            d16 = pl.ds(r16, 16)
            pltpu.async_copy(tabu_ref.at[:, :, pl.ds(ua, 16)],
                             stu_v.at[:, :, d16], sem_u)
            pltpu.async_copy(tabi_ref.at[:, :, pl.ds(ia, 16)],
                             sti_v.at[:, :, d16], sem_i)
            pltpu.async_copy(ub_ref.at[pl.ds(ua, 16)],
                             sub_v.at[d16], sem_ub)
            pltpu.async_copy(ib_ref.at[pl.ds(ia, 16)],
                             sib_v.at[d16], sem_ib)

            @pl.when(r >= _DEPTH)
            def _():
                drain_row(r - _DEPTH)

            return carry2

        lax.fori_loop(0, _BLKR, issue, 0)

        def tail(tr, carry2):
            drain_row(_BLKR - _DEPTH + tr)
            return carry2

        lax.fori_loop(0, _DEPTH, tail, 0)

        for g in range(_BLKR // 16):
            rvec = lanes + g * 16
            uvec = uix_v[pl.ds(b * _BLKR + g * 16, 16)]
            ivec = iix_v[pl.ds(b * _BLKR + g * 16, 16)]
            upos = rvec * 16 + (uvec & 15)
            ipos = rvec * 16 + (ivec & 15)
            for c in range(_EMB):
                cv = jnp.full((16,), c, jnp.int32)
                vu = plsc.load_gather(
                    stu_v, [cv >> 3, cv & 7, upos])
                plsc.store_scatter(x_w, [rvec + off, cv], vu)
                vi = plsc.load_gather(
                    sti_v, [cv >> 3, cv & 7, ipos])
                plsc.store_scatter(x_w, [rvec + off, cv + _EMB], vi)
            bs_w[pl.ds(off + g * 16, 16)] = (
                plsc.load_gather(sub_v, [upos])
                + plsc.load_gather(sib_v, [ipos]))

        pltpu.async_copy(x_w.at[pl.ds(off, _BLKR)], x_out.at[bs32], sem_ox)
        pltpu.async_copy(bs_w.at[pl.ds(off, _BLKR)], bs_out.at[bs32], sem_ob)
        return carry

    lax.fori_loop(0, nblk, block_body, 0)

    def out_drain(d, carry):
        dsl = pl.ds(d * _BLKR, _BLKR)
        pltpu.make_async_copy(x_w.at[dsl], x_out.at[pl.ds(base, _BLKR)],
                              sem_ox).wait()
        pltpu.make_async_copy(bs_w.at[dsl], bs_out.at[pl.ds(base, _BLKR)],
                              sem_ob).wait()
        return carry

    lax.fori_loop(0, 2, out_drain, 0)


def _make_sc_call(n):
    return functools.partial(
        pl.kernel,
        mesh=plsc.VectorSubcoreMesh(core_axis_name="c", subcore_axis_name="s"),
        compiler_params=pltpu.CompilerParams(needs_layout_passes=False),
        out_type=[
            jax.ShapeDtypeStruct((n, 2 * _EMB), jnp.float32),
            jax.ShapeDtypeStruct((n,), jnp.float32),
        ],
        scratch_types=[
            pltpu.VMEM((n // _NW,), jnp.int32),
            pltpu.VMEM((n // _NW,), jnp.int32),
            pltpu.VMEM((8, 8, 16 * _BLKR), jnp.float32),
            pltpu.VMEM((8, 8, 16 * _BLKR), jnp.float32),
            pltpu.VMEM((16 * _BLKR,), jnp.float32),
            pltpu.VMEM((16 * _BLKR,), jnp.float32),
            pltpu.VMEM((2 * _BLKR, 2 * _EMB), jnp.float32),
            pltpu.VMEM((2 * _BLKR,), jnp.float32),
            pltpu.SemaphoreType.DMA,
            pltpu.SemaphoreType.DMA,
            pltpu.SemaphoreType.DMA,
            pltpu.SemaphoreType.DMA,
            pltpu.SemaphoreType.DMA,
            pltpu.SemaphoreType.DMA,
        ],
    )(_sc_gather)


_BLK = 2048


def _mlp_body(x_ref, bs_ref, w1_ref, b1_ref, w2t_ref, b2_ref, w3r_ref,
              gb3_ref, out_ref):
    h1 = jnp.maximum(
        jnp.dot(x_ref[...], w1_ref[...], preferred_element_type=jnp.float32)
        + b1_ref[...], 0.0)
    h2 = jnp.maximum(
        lax.dot_general(h1, w2t_ref[...], (((1,), (1,)), ((), ())),
                        preferred_element_type=jnp.float32)
        + b2_ref[...], 0.0)
    m = jnp.sum(h2 * w3r_ref[...], axis=1)
    out_ref[...] = m + bs_ref[...] + gb3_ref[...]


def _mlp(x, bs, w1, b1, w2t, b2, w3r, gb3):
    grid = (x.shape[0] // _BLK,)
    return pl.pallas_call(
        _mlp_body,
        grid=grid,
        in_specs=[
            pl.BlockSpec((_BLK, 2 * _EMB), lambda i: (i, 0)),
            pl.BlockSpec((_BLK,), lambda i: (i,)),
            pl.BlockSpec((2 * _EMB, 128), lambda i: (0, 0)),
            pl.BlockSpec((128,), lambda i: (0,)),
            pl.BlockSpec((_EMB, 128), lambda i: (0, 0)),
            pl.BlockSpec((_EMB,), lambda i: (0,)),
            pl.BlockSpec((1, _EMB), lambda i: (0, 0)),
            pl.BlockSpec((1,), lambda i: (0,)),
        ],
        out_specs=pl.BlockSpec((_BLK,), lambda i: (i,)),
        out_shape=jax.ShapeDtypeStruct((x.shape[0],), jnp.float32),
    )(x, bs, w1, b1, w2t, b2, w3r, gb3)


def kernel(user_ids, item_ids, user_emb, item_emb, user_bias_t, item_bias_t,
           global_bias, W1, b1, W2, b2, W3, b3):
    uid = user_ids.astype(jnp.int32)
    iid = item_ids.astype(jnp.int32)
    tabu = user_emb.T.reshape(8, 8, 1000000)
    tabi = item_emb.T.reshape(8, 8, 1000000)

    w2t = W2.T
    w3r = W3.reshape(1, -1)
    gb3 = global_bias + b3

    x, bs = _make_sc_call(_B)(
        uid, iid, tabu, tabi,
        user_bias_t.reshape(-1), item_bias_t.reshape(-1))
    return _mlp(x, bs, W1, b1, w2t, b2, w3r, gb3)


# speedup vs baseline: 1.1276x; 1.0734x over previous
import functools

import jax
import jax.numpy as jnp
from jax import lax
from jax.experimental import pallas as pl
from jax.experimental.pallas import tpu as pltpu
from jax.experimental.pallas import tpu_sc as plsc

_EMB = 64
_B = 16384
_NC = 2
_NS = 16
_NW = _NC * _NS
_BPW = _B // _NW
_BLKR = 16


def _sc_gather(uid_ref, iid_ref, tabu_ref, tabi_ref, ub_ref, ib_ref,
               x_out, bs_out,
               uix_v, iix_v, stu_v, sti_v, sub_v, sib_v, x_w, bs_w,
               sem_u, sem_i, sem_ub, sem_ib, sem_ox, sem_ob):
    bpw = uix_v.shape[0]
    nblk = bpw // _BLKR
    wid = lax.axis_index("s") * _NC + lax.axis_index("c")
    base = wid * bpw

    for j in range(bpw // 128):
        sl = pl.ds(base + j * 128, 128)
        pltpu.sync_copy(uid_ref.at[sl], uix_v.at[pl.ds(j * 128, 128)])
        pltpu.sync_copy(iid_ref.at[sl], iix_v.at[pl.ds(j * 128, 128)])

    lanes = lax.iota(jnp.int32, 16)

    def issue_block(b):
        sbase = (b % 2) * (16 * _BLKR)
        uvec0 = uix_v[pl.ds(b * _BLKR, 16)]
        ivec0 = iix_v[pl.ds(b * _BLKR, 16)]

        def issue(r, carry):
            urid = jnp.sum(jnp.where(lanes == r, uvec0, 0))
            irid = jnp.sum(jnp.where(lanes == r, ivec0, 0))
            ua = pl.multiple_of((urid >> 4) << 4, 16)
            ia = pl.multiple_of((irid >> 4) << 4, 16)
            r16 = pl.multiple_of(sbase + r * 16, 16)
            d16 = pl.ds(r16, 16)
            pltpu.async_copy(tabu_ref.at[:, :, pl.ds(ua, 16)],
                             stu_v.at[:, :, d16], sem_u)
            pltpu.async_copy(tabi_ref.at[:, :, pl.ds(ia, 16)],
                             sti_v.at[:, :, d16], sem_i)
            pltpu.async_copy(ub_ref.at[pl.ds(ua, 16)],
                             sub_v.at[d16], sem_ub)
            pltpu.async_copy(ib_ref.at[pl.ds(ia, 16)],
                             sib_v.at[d16], sem_ib)
            return carry

        lax.fori_loop(0, _BLKR, issue, 0)

    def drain_block():
        nw = 16 * _BLKR
        pltpu.make_async_copy(tabu_ref.at[:, :, pl.ds(0, nw)],
                              stu_v.at[:, :, pl.ds(0, nw)], sem_u).wait()
        pltpu.make_async_copy(tabi_ref.at[:, :, pl.ds(0, nw)],
                              sti_v.at[:, :, pl.ds(0, nw)], sem_i).wait()
        pltpu.make_async_copy(ub_ref.at[pl.ds(0, nw)],
                              sub_v.at[pl.ds(0, nw)], sem_ub).wait()
        pltpu.make_async_copy(ib_ref.at[pl.ds(0, nw)],
                              sib_v.at[pl.ds(0, nw)], sem_ib).wait()

    def drain_out():
        pltpu.make_async_copy(x_w.at[pl.ds(0, _BLKR)],
                              x_out.at[pl.ds(base, _BLKR)], sem_ox).wait()
        pltpu.make_async_copy(bs_w.at[pl.ds(0, _BLKR)],
                              bs_out.at[pl.ds(base, _BLKR)], sem_ob).wait()

    def extract_block(m):
        sprev = (m % 2) * (16 * _BLKR)
        xoff = (m % 2) * _BLKR
        uvec = uix_v[pl.ds(m * _BLKR, 16)]
        ivec = iix_v[pl.ds(m * _BLKR, 16)]
        upos = sprev + lanes * 16 + (uvec & 15)
        ipos = sprev + lanes * 16 + (ivec & 15)
        for c in range(_EMB):
            cv = jnp.full((16,), c, jnp.int32)
            vu = plsc.load_gather(stu_v, [cv >> 3, cv & 7, upos])
            plsc.store_scatter(x_w, [lanes + xoff, cv], vu)
            vi = plsc.load_gather(sti_v, [cv >> 3, cv & 7, ipos])
            plsc.store_scatter(x_w, [lanes + xoff, cv + _EMB], vi)
        bs_w[pl.ds(xoff, 16)] = (plsc.load_gather(sub_v, [upos])
                                 + plsc.load_gather(sib_v, [ipos]))
        m16 = pl.ds(base + m * _BLKR, _BLKR)
        pltpu.async_copy(x_w.at[pl.ds(xoff, _BLKR)], x_out.at[m16], sem_ox)
        pltpu.async_copy(bs_w.at[pl.ds(xoff, _BLKR)], bs_out.at[m16], sem_ob)

    def block_body(b, carry):
        issue_block(b)

        @pl.when(b >= 1)
        def _():
            @pl.when(b >= 3)
            def _():
                drain_out()

            drain_block()
            extract_block(b - 1)

        return carry

    lax.fori_loop(0, nblk, block_body, 0)

    drain_block()
    drain_out()
    extract_block(nblk - 1)

    def tail_outs(d, carry):
        drain_out()
        return carry

    lax.fori_loop(0, 2, tail_outs, 0)


def _make_sc_call(n):
    return functools.partial(
        pl.kernel,
        mesh=plsc.VectorSubcoreMesh(core_axis_name="c", subcore_axis_name="s"),
        compiler_params=pltpu.CompilerParams(needs_layout_passes=False),
        out_type=[
            jax.ShapeDtypeStruct((n, 2 * _EMB), jnp.float32),
            jax.ShapeDtypeStruct((n,), jnp.float32),
        ],
        scratch_types=[
            pltpu.VMEM((n // _NW,), jnp.int32),
            pltpu.VMEM((n // _NW,), jnp.int32),
            pltpu.VMEM((8, 8, 32 * _BLKR), jnp.float32),
            pltpu.VMEM((8, 8, 32 * _BLKR), jnp.float32),
            pltpu.VMEM((32 * _BLKR,), jnp.float32),
            pltpu.VMEM((32 * _BLKR,), jnp.float32),
            pltpu.VMEM((2 * _BLKR, 2 * _EMB), jnp.float32),
            pltpu.VMEM((2 * _BLKR,), jnp.float32),
            pltpu.SemaphoreType.DMA,
            pltpu.SemaphoreType.DMA,
            pltpu.SemaphoreType.DMA,
            pltpu.SemaphoreType.DMA,
            pltpu.SemaphoreType.DMA,
            pltpu.SemaphoreType.DMA,
        ],
    )(_sc_gather)


_BLK = 2048


def _mlp_body(x_ref, bs_ref, w1_ref, b1_ref, w2t_ref, b2_ref, w3r_ref,
              gb3_ref, out_ref):
    h1 = jnp.maximum(
        jnp.dot(x_ref[...], w1_ref[...], preferred_element_type=jnp.float32)
        + b1_ref[...], 0.0)
    h2 = jnp.maximum(
        lax.dot_general(h1, w2t_ref[...], (((1,), (1,)), ((), ())),
                        preferred_element_type=jnp.float32)
        + b2_ref[...], 0.0)
    m = jnp.sum(h2 * w3r_ref[...], axis=1)
    out_ref[...] = m + bs_ref[...] + gb3_ref[...]


def _mlp(x, bs, w1, b1, w2t, b2, w3r, gb3):
    grid = (x.shape[0] // _BLK,)
    return pl.pallas_call(
        _mlp_body,
        grid=grid,
        in_specs=[
            pl.BlockSpec((_BLK, 2 * _EMB), lambda i: (i, 0)),
            pl.BlockSpec((_BLK,), lambda i: (i,)),
            pl.BlockSpec((2 * _EMB, 128), lambda i: (0, 0)),
            pl.BlockSpec((128,), lambda i: (0,)),
            pl.BlockSpec((_EMB, 128), lambda i: (0, 0)),
            pl.BlockSpec((_EMB,), lambda i: (0,)),
            pl.BlockSpec((1, _EMB), lambda i: (0, 0)),
            pl.BlockSpec((1,), lambda i: (0,)),
        ],
        out_specs=pl.BlockSpec((_BLK,), lambda i: (i,)),
        out_shape=jax.ShapeDtypeStruct((x.shape[0],), jnp.float32),
    )(x, bs, w1, b1, w2t, b2, w3r, gb3)


def kernel(user_ids, item_ids, user_emb, item_emb, user_bias_t, item_bias_t,
           global_bias, W1, b1, W2, b2, W3, b3):
    uid = user_ids.astype(jnp.int32)
    iid = item_ids.astype(jnp.int32)
    tabu = user_emb.T.reshape(8, 8, 1000000)
    tabi = item_emb.T.reshape(8, 8, 1000000)

    w2t = W2.T
    w3r = W3.reshape(1, -1)
    gb3 = global_bias + b3

    x, bs = _make_sc_call(_B)(
        uid, iid, tabu, tabi,
        user_bias_t.reshape(-1), item_bias_t.reshape(-1))
    return _mlp(x, bs, W1, b1, w2t, b2, w3r, gb3)


# speedup vs baseline: 1.1300x; 1.0022x over previous
import functools

import jax
import jax.numpy as jnp
from jax import lax
from jax.experimental import pallas as pl
from jax.experimental.pallas import tpu as pltpu
from jax.experimental.pallas import tpu_sc as plsc

_EMB = 64
_B = 16384
_NC = 2
_NS = 16
_NW = _NC * _NS
_BLKR = 16


def _sc_gather(uid_ref, iid_ref, tabu_ref, tabi_ref, ub_ref, ib_ref,
               x_out, bs_out,
               uix_v, iix_v, stu_v, sti_v, sub_v, sib_v, x_w, bs_w,
               sem_u, sem_i, sem_ub, sem_ib, sem_ox, sem_ob):
    bpw = uix_v.shape[0]
    nblk = bpw // _BLKR
    wid = lax.axis_index("s") * _NC + lax.axis_index("c")
    base = wid * bpw

    for j in range(bpw // 128):
        sl = pl.ds(base + j * 128, 128)
        pltpu.sync_copy(uid_ref.at[sl], uix_v.at[pl.ds(j * 128, 128)])
        pltpu.sync_copy(iid_ref.at[sl], iix_v.at[pl.ds(j * 128, 128)])

    lanes = lax.iota(jnp.int32, 16)

    def issue_block(b):
        sbase = (b % 2) * (16 * _BLKR)
        uvec0 = uix_v[pl.ds(b * _BLKR, 16)]
        ivec0 = iix_v[pl.ds(b * _BLKR, 16)]

        def issue(r, carry):
            urid = jnp.sum(jnp.where(lanes == r, uvec0, 0))
            irid = jnp.sum(jnp.where(lanes == r, ivec0, 0))
            ua = pl.multiple_of((urid >> 4) << 4, 16)
            ia = pl.multiple_of((irid >> 4) << 4, 16)
            r16 = pl.multiple_of(sbase + r * 16, 16)
            d16 = pl.ds(r16, 16)
            pltpu.async_copy(tabu_ref.at[:, :, pl.ds(ua, 16)],
                             stu_v.at[:, :, d16], sem_u)
            pltpu.async_copy(tabi_ref.at[:, :, pl.ds(ia, 16)],
                             sti_v.at[:, :, d16], sem_i)
            pltpu.async_copy(ub_ref.at[pl.ds(ua, 16)],
                             sub_v.at[d16], sem_ub)
            pltpu.async_copy(ib_ref.at[pl.ds(ia, 16)],
                             sib_v.at[d16], sem_ib)
            return carry

        lax.fori_loop(0, _BLKR, issue, 0)

    def drain_block():
        nw = 16 * _BLKR
        pltpu.make_async_copy(tabu_ref.at[:, :, pl.ds(0, nw)],
                              stu_v.at[:, :, pl.ds(0, nw)], sem_u).wait()
        pltpu.make_async_copy(tabi_ref.at[:, :, pl.ds(0, nw)],
                              sti_v.at[:, :, pl.ds(0, nw)], sem_i).wait()
        pltpu.make_async_copy(ub_ref.at[pl.ds(0, nw)],
                              sub_v.at[pl.ds(0, nw)], sem_ub).wait()
        pltpu.make_async_copy(ib_ref.at[pl.ds(0, nw)],
                              sib_v.at[pl.ds(0, nw)], sem_ib).wait()

    def drain_out():
        pltpu.make_async_copy(x_w.at[pl.ds(0, _BLKR)],
                              x_out.at[pl.ds(base, _BLKR)], sem_ox).wait()
        pltpu.make_async_copy(bs_w.at[pl.ds(0, _BLKR)],
                              bs_out.at[pl.ds(base, _BLKR)], sem_ob).wait()

    def extract_block(m):
        sprev = (m % 2) * (16 * _BLKR)
        xoff = (m % 2) * _BLKR
        uvec = uix_v[pl.ds(m * _BLKR, 16)]
        ivec = iix_v[pl.ds(m * _BLKR, 16)]
        upos = sprev + lanes * 16 + (uvec & 15)
        ipos = sprev + lanes * 16 + (ivec & 15)
        for c in range(_EMB):
            cv = jnp.full((16,), c, jnp.int32)
            vu = plsc.load_gather(stu_v, [cv >> 3, cv & 7, upos])
            plsc.store_scatter(x_w, [lanes + xoff, cv], vu)
            vi = plsc.load_gather(sti_v, [cv >> 3, cv & 7, ipos])
            plsc.store_scatter(x_w, [lanes + xoff, cv + _EMB], vi)
        bs_w[pl.ds(xoff, 16)] = (plsc.load_gather(sub_v, [upos])
                                 + plsc.load_gather(sib_v, [ipos]))
        m16 = pl.ds(base + m * _BLKR, _BLKR)
        pltpu.async_copy(x_w.at[pl.ds(xoff, _BLKR)], x_out.at[m16], sem_ox)
        pltpu.async_copy(bs_w.at[pl.ds(xoff, _BLKR)], bs_out.at[m16], sem_ob)

    def block_body(b, carry):
        issue_block(b)

        @pl.when(b >= 1)
        def _():
            @pl.when(b >= 3)
            def _():
                drain_out()

            drain_block()
            extract_block(b - 1)

        return carry

    lax.fori_loop(0, nblk, block_body, 0)

    drain_block()
    drain_out()
    extract_block(nblk - 1)

    def tail_outs(d, carry):
        drain_out()
        return carry

    lax.fori_loop(0, 2, tail_outs, 0)


def _make_sc_call(n):
    return functools.partial(
        pl.kernel,
        mesh=plsc.VectorSubcoreMesh(core_axis_name="c", subcore_axis_name="s"),
        compiler_params=pltpu.CompilerParams(needs_layout_passes=False),
        out_type=[
            jax.ShapeDtypeStruct((n, 2 * _EMB), jnp.float32),
            jax.ShapeDtypeStruct((n,), jnp.float32),
        ],
        scratch_types=[
            pltpu.VMEM((n // _NW,), jnp.int32),
            pltpu.VMEM((n // _NW,), jnp.int32),
            pltpu.VMEM((8, 8, 32 * _BLKR), jnp.float32),
            pltpu.VMEM((8, 8, 32 * _BLKR), jnp.float32),
            pltpu.VMEM((32 * _BLKR,), jnp.float32),
            pltpu.VMEM((32 * _BLKR,), jnp.float32),
            pltpu.VMEM((2 * _BLKR, 2 * _EMB), jnp.float32),
            pltpu.VMEM((2 * _BLKR,), jnp.float32),
            pltpu.SemaphoreType.DMA,
            pltpu.SemaphoreType.DMA,
            pltpu.SemaphoreType.DMA,
            pltpu.SemaphoreType.DMA,
            pltpu.SemaphoreType.DMA,
            pltpu.SemaphoreType.DMA,
        ],
    )(_sc_gather)


_BLK = 2048


def _mlp_body(x_ref, bs_ref, w1_ref, b1_ref, w2t_ref, b2_ref, w3r_ref,
              gb3_ref, out_ref):
    h1 = jnp.maximum(
        jnp.dot(x_ref[...], w1_ref[...], preferred_element_type=jnp.float32)
        + b1_ref[...], 0.0)
    h2 = jnp.maximum(
        lax.dot_general(h1, w2t_ref[...], (((1,), (1,)), ((), ())),
                        preferred_element_type=jnp.float32)
        + b2_ref[...], 0.0)
    m = jnp.sum(h2 * w3r_ref[...], axis=1)
    out_ref[...] = m + bs_ref[...] + gb3_ref[...]


def _mlp(x, bs, w1, b1, w2t, b2, w3r, gb3):
    grid = (x.shape[0] // _BLK,)
    return pl.pallas_call(
        _mlp_body,
        grid=grid,
        in_specs=[
            pl.BlockSpec((_BLK, 2 * _EMB), lambda i: (i, 0)),
            pl.BlockSpec((_BLK,), lambda i: (i,)),
            pl.BlockSpec((2 * _EMB, 128), lambda i: (0, 0)),
            pl.BlockSpec((128,), lambda i: (0,)),
            pl.BlockSpec((_EMB, 128), lambda i: (0, 0)),
            pl.BlockSpec((_EMB,), lambda i: (0,)),
            pl.BlockSpec((1, _EMB), lambda i: (0, 0)),
            pl.BlockSpec((1,), lambda i: (0,)),
        ],
        out_specs=pl.BlockSpec((_BLK,), lambda i: (i,)),
        out_shape=jax.ShapeDtypeStruct((x.shape[0],), jnp.float32),
    )(x, bs, w1, b1, w2t, b2, w3r, gb3)


def kernel(user_ids, item_ids, user_emb, item_emb, user_bias_t, item_bias_t,
           global_bias, W1, b1, W2, b2, W3, b3):
    uid = user_ids.astype(jnp.int32)
    iid = item_ids.astype(jnp.int32)
    tabu = user_emb.T.reshape(8, 8, 1000000)
    tabi = item_emb.T.reshape(8, 8, 1000000)

    w2t = W2.T
    w3r = W3.reshape(1, -1)
    gb3 = global_bias + b3

    x, bs = _make_sc_call(_B)(
        uid, iid, tabu, tabi,
        user_bias_t.reshape(-1), item_bias_t.reshape(-1))
    return _mlp(x, bs, W1, b1, w2t, b2, w3r, gb3)
